# Initial kernel scaffold; baseline (speedup 1.0000x reference)
import jax
import jax.numpy as jnp
from jax.experimental import pallas as pl


def kernel(x, weekdays, hours, pos_table, weekday_table, hour_table, decay):
    raise NotImplementedError("write your pallas kernel here")


# speedup vs baseline: 6.9315x; 6.9315x over previous
import functools

import jax
import jax.numpy as jnp
from jax import lax
from jax.experimental import pallas as pl
from jax.experimental.pallas import tpu as pltpu

B_BLK = 8


def _body(x_ref, w_ref, h_ref, pos_ref, wk_ref, hr_ref, decay_ref, out_ref):
    seq = x_ref.shape[1]
    d4 = wk_ref.shape[1]

    s_iota = lax.broadcasted_iota(jnp.int32, (seq, 1), 0).astype(jnp.float32)
    dw = jnp.exp(-decay_ref[0] * (seq - 1 - s_iota) / seq)
    pos = pos_ref[:] * dw

    w_idx = w_ref[:][:, :, None]
    h_idx = h_ref[:][:, :, None]
    wk_emb = jnp.zeros((x_ref.shape[0], seq, d4), jnp.float32)
    for k in range(wk_ref.shape[0]):
        wk_emb = jnp.where(w_idx == k, wk_ref[k, :][None, None, :], wk_emb)
    hr_emb = jnp.zeros((x_ref.shape[0], seq, d4), jnp.float32)
    for k in range(hr_ref.shape[0]):
        hr_emb = jnp.where(h_idx == k, hr_ref[k, :][None, None, :], hr_emb)

    x = x_ref[:]
    out_ref[:] = jnp.concatenate(
        [x[:, :, :64] + pos[None, :, :],
         x[:, :, 64:96] + wk_emb,
         x[:, :, 96:128] + hr_emb], axis=-1)


@jax.jit
def kernel(x, weekdays, hours, pos_table, weekday_table, hour_table, decay):
    batch, seq, d_model = x.shape
    grid = (batch // B_BLK,)
    return pl.pallas_call(
        _body,
        grid=grid,
        in_specs=[
            pl.BlockSpec((B_BLK, seq, d_model), lambda i: (i, 0, 0)),
            pl.BlockSpec((B_BLK, seq), lambda i: (i, 0)),
            pl.BlockSpec((B_BLK, seq), lambda i: (i, 0)),
            pl.BlockSpec(pos_table.shape, lambda i: (0, 0)),
            pl.BlockSpec(weekday_table.shape, lambda i: (0, 0)),
            pl.BlockSpec(hour_table.shape, lambda i: (0, 0)),
            pl.BlockSpec(memory_space=pltpu.SMEM),
        ],
        out_specs=pl.BlockSpec((B_BLK, seq, d_model), lambda i: (i, 0, 0)),
        out_shape=jax.ShapeDtypeStruct(x.shape, x.dtype),
    )(x, weekdays, hours, pos_table, weekday_table, hour_table, decay)


# speedup vs baseline: 16.6380x; 2.4003x over previous
import jax
import jax.numpy as jnp
from jax import lax
from jax.experimental import pallas as pl
from jax.experimental.pallas import tpu as pltpu

B_BLK = 8


def _body(x_ref, w_ref, h_ref, pos_ref, tbl_ref, decay_ref, out_ref):
    bblk, seq, d_model = x_ref.shape
    n = bblk * seq

    s_iota = lax.broadcasted_iota(jnp.int32, (seq, 1), 0).astype(jnp.float32)
    dw = jnp.exp(-decay_ref[0] * (seq - 1 - s_iota) / seq)
    pos_pad = jnp.concatenate(
        [pos_ref[:] * dw, jnp.zeros((seq, d_model - pos_ref.shape[1]),
                                    jnp.float32)], axis=-1)

    lane = lax.broadcasted_iota(jnp.int32, (bblk, seq, 64), 2)
    w_idx = w_ref[:][:, :, None]
    h_idx = h_ref[:][:, :, None] + 32
    onehot = (jnp.where(lane == w_idx, 1.0, 0.0)
              + jnp.where(lane == h_idx, 1.0, 0.0)).astype(jnp.bfloat16)
    emb = lax.dot_general(
        onehot.reshape(n, 64), tbl_ref[:],
        dimension_numbers=(((1,), (0,)), ((), ())),
        preferred_element_type=jnp.float32).reshape(bblk, seq, d_model)

    out_ref[:] = x_ref[:] + emb + pos_pad[None, :, :]


@jax.jit
def kernel(x, weekdays, hours, pos_table, weekday_table, hour_table, decay):
    batch, seq, d_model = x.shape
    d4 = weekday_table.shape[1]

    tbl = jnp.zeros((64, d_model), jnp.float32)
    tbl = tbl.at[:weekday_table.shape[0], 2 * d4:3 * d4].set(weekday_table)
    tbl = tbl.at[32:32 + hour_table.shape[0], 3 * d4:].set(hour_table)
    tbl = tbl.astype(jnp.bfloat16)

    grid = (batch // B_BLK,)
    return pl.pallas_call(
        _body,
        grid=grid,
        in_specs=[
            pl.BlockSpec((B_BLK, seq, d_model), lambda i: (i, 0, 0)),
            pl.BlockSpec((B_BLK, seq), lambda i: (i, 0)),
            pl.BlockSpec((B_BLK, seq), lambda i: (i, 0)),
            pl.BlockSpec(pos_table.shape, lambda i: (0, 0)),
            pl.BlockSpec(tbl.shape, lambda i: (0, 0)),
            pl.BlockSpec(memory_space=pltpu.SMEM),
        ],
        out_specs=pl.BlockSpec((B_BLK, seq, d_model), lambda i: (i, 0, 0)),
        out_shape=jax.ShapeDtypeStruct(x.shape, x.dtype),
    )(x, weekdays, hours, pos_table, tbl, decay)


# speedup vs baseline: 20.1014x; 1.2082x over previous
import jax
import jax.numpy as jnp
from jax import lax
from jax.experimental import pallas as pl
from jax.experimental.pallas import tpu as pltpu

B_BLK = 16


def _body(x_ref, w_ref, h_ref, pos_ref, tbl_ref, decay_ref, out_ref):
    bblk, seq, d_model = x_ref.shape
    n = bblk * seq

    s_iota = lax.broadcasted_iota(jnp.int32, (seq, 1), 0).astype(jnp.float32)
    dw = jnp.exp(-decay_ref[0] * (seq - 1 - s_iota) / seq)
    pos = pos_ref[:] * dw

    lane = lax.broadcasted_iota(jnp.int32, (bblk, seq, 64), 2)
    sel_idx = jnp.where(lane < 32, w_ref[:][:, :, None],
                        h_ref[:][:, :, None] + 32)
    onehot = jnp.where(lane == sel_idx, 1.0, 0.0).astype(jnp.bfloat16)
    emb = lax.dot_general(
        onehot.reshape(n, 64), tbl_ref[:],
        dimension_numbers=(((1,), (0,)), ((), ())),
        preferred_element_type=jnp.float32).reshape(bblk, seq, d_model)

    out_ref[:] = x_ref[:] + emb + pos[None, :, :]


@jax.jit
def kernel(x, weekdays, hours, pos_table, weekday_table, hour_table, decay):
    batch, seq, d_model = x.shape
    d4 = weekday_table.shape[1]

    tbl = jnp.zeros((64, d_model), jnp.float32)
    tbl = tbl.at[:weekday_table.shape[0], 2 * d4:3 * d4].set(weekday_table)
    tbl = tbl.at[32:32 + hour_table.shape[0], 3 * d4:].set(hour_table)
    tbl = tbl.astype(jnp.bfloat16)
    pos_pad = jnp.pad(pos_table, ((0, 0), (0, d_model - pos_table.shape[1])))

    grid = (batch // B_BLK,)
    return pl.pallas_call(
        _body,
        grid=grid,
        in_specs=[
            pl.BlockSpec((B_BLK, seq, d_model), lambda i: (i, 0, 0)),
            pl.BlockSpec((B_BLK, seq), lambda i: (i, 0)),
            pl.BlockSpec((B_BLK, seq), lambda i: (i, 0)),
            pl.BlockSpec(pos_pad.shape, lambda i: (0, 0)),
            pl.BlockSpec(tbl.shape, lambda i: (0, 0)),
            pl.BlockSpec(memory_space=pltpu.SMEM),
        ],
        out_specs=pl.BlockSpec((B_BLK, seq, d_model), lambda i: (i, 0, 0)),
        out_shape=jax.ShapeDtypeStruct(x.shape, x.dtype),
    )(x, weekdays, hours, pos_pad, tbl, decay)
